# baseline (speedup 1.0000x reference)
import functools

import jax
import jax.numpy as jnp
from jax import lax
from jax.experimental import pallas as pl
from jax.experimental.pallas import tpu as pltpu
from jax.experimental.pallas import tpu_sc as plsc

N = 10000
E = 320000
D = 128
H = 128
C = 40
CP = 128
NPAD = 10240
NC = 2
NS = 16
NW = NC * NS
LANES = 16

KD = 80
DCH = E // KD
KE = 125
ROWS_PER_TILE = NPAD // NS

_MESH = plsc.VectorSubcoreMesh(
    core_axis_name="c", subcore_axis_name="s", num_cores=NC, num_subcores=NS
)


def _zero_fill_2d(ref, rows, cols):
    zv = jnp.zeros((LANES,), jnp.float32)

    def body(i, _):
        r = i // (cols // LANES)
        col = (i % (cols // LANES)) * LANES
        ref[r, pl.ds(col, LANES)] = zv
        return 0

    lax.fori_loop(0, rows * (cols // LANES), body, 0)


@functools.partial(
    pl.kernel,
    out_type=jax.ShapeDtypeStruct((NC, 1, NPAD), jnp.float32),
    mesh=_MESH,
    scratch_types=[
        pltpu.VMEM_SHARED((NPAD,), jnp.float32),
        pltpu.VMEM((DCH // NS, KD), jnp.int32),
        pltpu.VMEM((KD,), jnp.float32),
        pltpu.VMEM((ROWS_PER_TILE,), jnp.float32),
        pltpu.SemaphoreType.DMA,
        pltpu.SemaphoreType.DMA,
    ],
)
def _degrees(ei_hbm, deg_hbm, acc, idx_v, ones_v, zb_v, sa, sb):
    c = lax.axis_index("c")
    s = lax.axis_index("s")

    def fill(i, _):
        ones_v[pl.ds(i * LANES, LANES)] = jnp.ones((LANES,), jnp.float32)
        return 0

    lax.fori_loop(0, KD // LANES, fill, 0)

    def fillz(i, _):
        zb_v[pl.ds(i * LANES, LANES)] = jnp.zeros((LANES,), jnp.float32)
        return 0

    lax.fori_loop(0, ROWS_PER_TILE // LANES, fillz, 0)

    pltpu.sync_copy(zb_v, acc.at[pl.ds(s * ROWS_PER_TILE, ROWS_PER_TILE)])
    plsc.subcore_barrier()

    nrows = DCH // NS
    pltpu.sync_copy(ei_hbm.at[c, s], idx_v)

    def sca_d(j, sem):
        return pltpu.make_async_copy(ones_v, acc.at[idx_v.at[j]], sem)

    sca_d(0, sa).start(add=True)

    def chunk(i2, _):
        j = 2 * i2 + 1
        sca_d(j, sb).start(add=True)
        sca_d(j - 1, sa).wait()
        sca_d(j + 1, sa).start(add=True)
        sca_d(j, sb).wait()
        return 0

    lax.fori_loop(0, (nrows - 2) // 2, chunk, 0)
    sca_d(nrows - 1, sb).start(add=True)
    sca_d(nrows - 2, sa).wait()
    sca_d(nrows - 1, sb).wait()
    plsc.subcore_barrier()

    pltpu.sync_copy(
        acc.at[pl.ds(s * ROWS_PER_TILE, ROWS_PER_TILE)],
        deg_hbm.at[c, 0, pl.ds(s * ROWS_PER_TILE, ROWS_PER_TILE)],
    )


NCHK = (E // NW) // KE
ZROWS = 40


def _make_agg(width):
    @functools.partial(
        pl.kernel,
        out_type=jax.ShapeDtypeStruct((NC, NPAD, width), jnp.float32),
        mesh=_MESH,
        scratch_types=[
            pltpu.VMEM_SHARED((NPAD, width), jnp.float32),
            pltpu.VMEM((KE, width), jnp.float32),
            pltpu.VMEM((KE, width), jnp.float32),
            pltpu.VMEM((2, KE), jnp.int32),
            pltpu.VMEM((2, KE), jnp.int32),
            pltpu.VMEM((2, KE), jnp.int32),
            pltpu.VMEM((2, KE), jnp.int32),
            pltpu.SemaphoreType.DMA,
            pltpu.SemaphoreType.DMA,
            pltpu.SemaphoreType.DMA,
            pltpu.SemaphoreType.DMA,
            pltpu.SemaphoreType.DMA,
        ],
    )
    def agg(h_hbm, eidx_hbm, out_hbm, acc, r0, r1, i0, i1, i2, i3,
            sg0, sg1, ss0, ss1, si):
        c = lax.axis_index("c")
        s = lax.axis_index("s")
        wid = c * NS + s
        rr = (r0, r1)
        ii = (i0, i1, i2, i3)
        sg = (sg0, sg1)
        ss = (ss0, ss1)

        def gat(iv, rv, sem):
            return pltpu.make_async_copy(h_hbm.at[iv.at[0]], rv, sem)

        def sca(rv, iv, sem):
            return pltpu.make_async_copy(rv, acc.at[iv.at[1]], sem)

        def ipre(j, iv):
            pltpu.make_async_copy(eidx_hbm.at[wid * NCHK + j], iv, si).start()

        def iwait(iv):
            pltpu.make_async_copy(eidx_hbm.at[0], iv, si).wait()

        _zero_fill_2d(r0, ZROWS, width)

        def zc(k, _):
            pltpu.sync_copy(
                r0.at[pl.ds(0, ZROWS), :],
                acc.at[pl.ds(s * ROWS_PER_TILE + k * ZROWS, ZROWS), :],
            )
            return 0

        lax.fori_loop(0, ROWS_PER_TILE // ZROWS, zc, 0)
        plsc.subcore_barrier()

        def emit(p, k, jn):
            iwait(ii[k])
            sca(rr[p], ii[k], ss[p]).wait()
            gat(ii[k], rr[p], sg[p]).start()
            gat(ii[k], rr[1 - p], sg[1 - p]).wait()
            sca(rr[1 - p], ii[(k - 1) % 4], ss[1 - p]).start(add=True)
            if jn is not None:
                ipre(jn, ii[(k + 1) % 4])

        pltpu.sync_copy(eidx_hbm.at[wid * NCHK], i0)
        gat(i0, r0, sg0).start()
        ipre(1, i1)
        iwait(i1)
        gat(i1, r1, sg1).start()
        gat(i1, r0, sg0).wait()
        sca(r0, i0, ss0).start(add=True)
        ipre(2, i2)

        def body(t, _):
            j = 4 * t + 2
            emit(0, 2, j + 1)
            emit(1, 3, j + 2)
            emit(0, 0, j + 3)
            emit(1, 1, j + 4)
            return 0

        lax.fori_loop(0, (NCHK - 4) // 4, body, 0)
        emit(0, 2, NCHK - 1)
        emit(1, 3, None)
        gat(i3, r1, sg1).wait()
        sca(r1, i3, ss1).start(add=True)
        sca(r0, i2, ss0).wait()
        sca(r1, i3, ss1).wait()
        plsc.subcore_barrier()

        pltpu.sync_copy(
            acc.at[pl.ds(s * ROWS_PER_TILE, ROWS_PER_TILE), :],
            out_hbm.at[c, pl.ds(s * ROWS_PER_TILE, ROWS_PER_TILE), :],
        )

    return agg


_agg_h = _make_agg(H)
_agg_c = _agg_h


_BLK = 1024
_GRID = NPAD // _BLK


def _mm_body(x_ref, w_ref, o_ref):
    o_ref[...] = jnp.dot(
        x_ref[...], w_ref[...], preferred_element_type=jnp.float32
    )


def _mm(xp, w):
    return pl.pallas_call(
        _mm_body,
        grid=(_GRID,),
        in_specs=[
            pl.BlockSpec((_BLK, D), lambda i: (i, 0)),
            pl.BlockSpec((D, H), lambda i: (0, 0)),
        ],
        out_specs=pl.BlockSpec((_BLK, H), lambda i: (i, 0)),
        out_shape=jax.ShapeDtypeStruct((NPAD, H), jnp.float32),
    )(xp, w)


def _scale_body(x_ref, dout_ref, o_ref):
    rs = lax.rsqrt(jnp.maximum(dout_ref[...], 1.0))
    o_ref[...] = x_ref[...] * rs


def _scale(xw, deg_out):
    return pl.pallas_call(
        _scale_body,
        grid=(_GRID,),
        in_specs=[
            pl.BlockSpec((_BLK, H), lambda i: (i, 0)),
            pl.BlockSpec((_BLK, 1), lambda i: (i, 0)),
        ],
        out_specs=pl.BlockSpec((_BLK, H), lambda i: (i, 0)),
        out_shape=jax.ShapeDtypeStruct((NPAD, H), jnp.float32),
    )(xw, deg_out)


def _combine_mm_body(p_ref, b_ref, din_ref, dout_ref, w_ref, o_ref):
    rs_in = lax.rsqrt(jnp.maximum(din_ref[...], 1.0))
    rs_out = lax.rsqrt(jnp.maximum(dout_ref[...], 1.0))
    h = (p_ref[0] + p_ref[1]) * rs_in + b_ref[...]
    h = jnp.maximum(h, 0.0)
    o_ref[...] = jnp.dot(
        h, w_ref[...], preferred_element_type=jnp.float32
    ) * rs_out


def _combine_mm(p, b, deg_in, deg_out, w):
    wout = w.shape[1]
    return pl.pallas_call(
        _combine_mm_body,
        grid=(_GRID,),
        in_specs=[
            pl.BlockSpec((NC, _BLK, H), lambda i: (0, i, 0)),
            pl.BlockSpec((1, H), lambda i: (0, 0)),
            pl.BlockSpec((_BLK, 1), lambda i: (i, 0)),
            pl.BlockSpec((_BLK, 1), lambda i: (i, 0)),
            pl.BlockSpec((H, wout), lambda i: (0, 0)),
        ],
        out_specs=pl.BlockSpec((_BLK, wout), lambda i: (i, 0)),
        out_shape=jax.ShapeDtypeStruct((NPAD, wout), jnp.float32),
    )(p, b, deg_in, deg_out, w)


def _final_body(p_ref, b_ref, din_ref, o_ref):
    rs_in = lax.rsqrt(jnp.maximum(din_ref[...], 1.0))
    o_ref[...] = (p_ref[0] + p_ref[1]) * rs_in + b_ref[...]


def _final(p, b, deg_in):
    return pl.pallas_call(
        _final_body,
        grid=(_GRID,),
        in_specs=[
            pl.BlockSpec((NC, _BLK, CP), lambda i: (0, i, 0)),
            pl.BlockSpec((1, CP), lambda i: (0, 0)),
            pl.BlockSpec((_BLK, 1), lambda i: (i, 0)),
        ],
        out_specs=pl.BlockSpec((_BLK, CP), lambda i: (i, 0)),
        out_shape=jax.ShapeDtypeStruct((NPAD, CP), jnp.float32),
    )(p, b, deg_in)


def kernel(x, edge_index, W1, b1, W2, b2, W3, b3):
    xp = jnp.zeros((NPAD, D), jnp.float32).at[:N].set(x)
    ei4 = edge_index.reshape(2, NS, DCH // NS, KD)
    eidx = jnp.stack(
        [
            edge_index[0].reshape(NW, NCHK, KE),
            edge_index[1].reshape(NW, NCHK, KE),
        ],
        axis=2,
    ).reshape(NW * NCHK, 2, KE)
    W3p = jnp.pad(W3, ((0, 0), (0, CP - C)))
    b3p = jnp.pad(b3, (0, CP - C))

    xw1 = _mm(xp, W1)
    deg = _degrees(ei4)
    deg_out = deg[0].reshape(NPAD, 1)
    deg_in = deg[1].reshape(NPAD, 1)

    h1 = _scale(xw1, deg_out)
    p1 = _agg_h(h1, eidx)
    h2 = _combine_mm(p1, b1.reshape(1, H), deg_in, deg_out, W2)
    p2 = _agg_h(h2, eidx)
    h3 = _combine_mm(p2, b2.reshape(1, H), deg_in, deg_out, W3p)
    p3 = _agg_c(h3, eidx)
    out = _final(p3, b3p.reshape(1, CP), deg_in)
    return out[:N, :C]


# speedup vs baseline: 1.0184x; 1.0184x over previous
import functools

import jax
import jax.numpy as jnp
from jax import lax
from jax.experimental import pallas as pl
from jax.experimental.pallas import tpu as pltpu
from jax.experimental.pallas import tpu_sc as plsc

N = 10000
E = 320000
D = 128
H = 128
C = 40
CP = 128
NPAD = 10240
NC = 2
NS = 16
NW = NC * NS
LANES = 16

KD = 80
DCH = E // KD
KE = 125
ROWS_PER_TILE = NPAD // NS

_MESH = plsc.VectorSubcoreMesh(
    core_axis_name="c", subcore_axis_name="s", num_cores=NC, num_subcores=NS
)


def _zero_fill_2d(ref, rows, cols):
    zv = jnp.zeros((LANES,), jnp.float32)

    def body(i, _):
        r = i // (cols // LANES)
        col = (i % (cols // LANES)) * LANES
        ref[r, pl.ds(col, LANES)] = zv
        return 0

    lax.fori_loop(0, rows * (cols // LANES), body, 0)


@functools.partial(
    pl.kernel,
    out_type=jax.ShapeDtypeStruct((NC, 1, NPAD), jnp.float32),
    mesh=_MESH,
    scratch_types=[
        pltpu.VMEM_SHARED((NPAD,), jnp.float32),
        pltpu.VMEM((DCH // NS, KD), jnp.int32),
        pltpu.VMEM((KD,), jnp.float32),
        pltpu.VMEM((ROWS_PER_TILE,), jnp.float32),
        pltpu.SemaphoreType.DMA,
        pltpu.SemaphoreType.DMA,
    ],
)
def _degrees(ei_hbm, deg_hbm, acc, idx_v, ones_v, zb_v, sa, sb):
    c = lax.axis_index("c")
    s = lax.axis_index("s")

    def fill(i, _):
        ones_v[pl.ds(i * LANES, LANES)] = jnp.ones((LANES,), jnp.float32)
        return 0

    lax.fori_loop(0, KD // LANES, fill, 0)

    def fillz(i, _):
        zb_v[pl.ds(i * LANES, LANES)] = jnp.zeros((LANES,), jnp.float32)
        return 0

    lax.fori_loop(0, ROWS_PER_TILE // LANES, fillz, 0)

    pltpu.sync_copy(zb_v, acc.at[pl.ds(s * ROWS_PER_TILE, ROWS_PER_TILE)])
    plsc.subcore_barrier()

    nrows = DCH // NS
    pltpu.sync_copy(ei_hbm.at[c, s], idx_v)

    def sca_d(j, sem):
        return pltpu.make_async_copy(ones_v, acc.at[idx_v.at[j]], sem)

    sca_d(0, sa).start(add=True)

    def chunk(i2, _):
        j = 2 * i2 + 1
        sca_d(j, sb).start(add=True)
        sca_d(j - 1, sa).wait()
        sca_d(j + 1, sa).start(add=True)
        sca_d(j, sb).wait()
        return 0

    lax.fori_loop(0, (nrows - 2) // 2, chunk, 0)
    sca_d(nrows - 1, sb).start(add=True)
    sca_d(nrows - 2, sa).wait()
    sca_d(nrows - 1, sb).wait()
    plsc.subcore_barrier()

    pltpu.sync_copy(
        acc.at[pl.ds(s * ROWS_PER_TILE, ROWS_PER_TILE)],
        deg_hbm.at[c, 0, pl.ds(s * ROWS_PER_TILE, ROWS_PER_TILE)],
    )


NCHK = (E // NW) // KE
ZROWS = 80


def _make_agg(width):
    @functools.partial(
        pl.kernel,
        out_type=jax.ShapeDtypeStruct((NC, NPAD, width), jnp.float32),
        mesh=_MESH,
        scratch_types=[
            pltpu.VMEM_SHARED((NPAD, width), jnp.float32),
            pltpu.VMEM((KE, width), jnp.float32),
            pltpu.VMEM((KE, width), jnp.float32),
            pltpu.VMEM((2, KE), jnp.int32),
            pltpu.VMEM((2, KE), jnp.int32),
            pltpu.VMEM((2, KE), jnp.int32),
            pltpu.VMEM((2, KE), jnp.int32),
            pltpu.SemaphoreType.DMA,
            pltpu.SemaphoreType.DMA,
            pltpu.SemaphoreType.DMA,
            pltpu.SemaphoreType.DMA,
            pltpu.SemaphoreType.DMA,
        ],
    )
    def agg(h_hbm, eidx_hbm, out_hbm, acc, r0, r1, i0, i1, i2, i3,
            sg0, sg1, ss0, ss1, si):
        c = lax.axis_index("c")
        s = lax.axis_index("s")
        wid = c * NS + s
        rr = (r0, r1)
        ii = (i0, i1, i2, i3)
        sg = (sg0, sg1)
        ss = (ss0, ss1)

        def gat(iv, rv, sem):
            return pltpu.make_async_copy(h_hbm.at[iv.at[0]], rv, sem)

        def sca(rv, iv, sem):
            return pltpu.make_async_copy(rv, acc.at[iv.at[1]], sem)

        def ipre(j, iv):
            pltpu.make_async_copy(eidx_hbm.at[wid * NCHK + j], iv, si).start()

        def iwait(iv):
            pltpu.make_async_copy(eidx_hbm.at[0], iv, si).wait()

        _zero_fill_2d(r0, ZROWS, width)

        def zc(k, _):
            pltpu.sync_copy(
                r0.at[pl.ds(0, ZROWS), :],
                acc.at[pl.ds(s * ROWS_PER_TILE + k * ZROWS, ZROWS), :],
            )
            return 0

        lax.fori_loop(0, ROWS_PER_TILE // ZROWS, zc, 0)
        plsc.subcore_barrier()

        def emit(p, k, jn):
            iwait(ii[k])
            sca(rr[p], ii[k], ss[p]).wait()
            gat(ii[k], rr[p], sg[p]).start()
            gat(ii[k], rr[1 - p], sg[1 - p]).wait()
            sca(rr[1 - p], ii[(k - 1) % 4], ss[1 - p]).start(add=True)
            if jn is not None:
                ipre(jn, ii[(k + 1) % 4])

        pltpu.sync_copy(eidx_hbm.at[wid * NCHK], i0)
        gat(i0, r0, sg0).start()
        ipre(1, i1)
        iwait(i1)
        gat(i1, r1, sg1).start()
        gat(i1, r0, sg0).wait()
        sca(r0, i0, ss0).start(add=True)
        ipre(2, i2)

        def body(t, _):
            j = 4 * t + 2
            emit(0, 2, j + 1)
            emit(1, 3, j + 2)
            emit(0, 0, j + 3)
            emit(1, 1, j + 4)
            return 0

        lax.fori_loop(0, (NCHK - 4) // 4, body, 0)
        emit(0, 2, NCHK - 1)
        emit(1, 3, None)
        gat(i3, r1, sg1).wait()
        sca(r1, i3, ss1).start(add=True)
        sca(r0, i2, ss0).wait()
        sca(r1, i3, ss1).wait()
        plsc.subcore_barrier()

        pltpu.sync_copy(
            acc.at[pl.ds(s * ROWS_PER_TILE, ROWS_PER_TILE), :],
            out_hbm.at[c, pl.ds(s * ROWS_PER_TILE, ROWS_PER_TILE), :],
        )

    return agg


_agg_h = _make_agg(H)
_agg_c = _agg_h


_BLK = 1024
_GRID = NPAD // _BLK


def _mm_scale_body(x_ref, w_ref, dout_ref, o_ref):
    rs = lax.rsqrt(jnp.maximum(dout_ref[...], 1.0))
    o_ref[...] = jnp.dot(
        x_ref[...], w_ref[...], preferred_element_type=jnp.float32
    ) * rs


def _mm_scale(xp, w, deg_out):
    return pl.pallas_call(
        _mm_scale_body,
        grid=(_GRID,),
        in_specs=[
            pl.BlockSpec((_BLK, D), lambda i: (i, 0)),
            pl.BlockSpec((D, H), lambda i: (0, 0)),
            pl.BlockSpec((_BLK, 1), lambda i: (i, 0)),
        ],
        out_specs=pl.BlockSpec((_BLK, H), lambda i: (i, 0)),
        out_shape=jax.ShapeDtypeStruct((NPAD, H), jnp.float32),
    )(xp, w, deg_out)


def _combine_mm_body(p_ref, b_ref, din_ref, dout_ref, w_ref, o_ref):
    rs_in = lax.rsqrt(jnp.maximum(din_ref[...], 1.0))
    rs_out = lax.rsqrt(jnp.maximum(dout_ref[...], 1.0))
    h = (p_ref[0] + p_ref[1]) * rs_in + b_ref[...]
    h = jnp.maximum(h, 0.0)
    o_ref[...] = jnp.dot(
        h, w_ref[...], preferred_element_type=jnp.float32
    ) * rs_out


def _combine_mm(p, b, deg_in, deg_out, w):
    wout = w.shape[1]
    return pl.pallas_call(
        _combine_mm_body,
        grid=(_GRID,),
        in_specs=[
            pl.BlockSpec((NC, _BLK, H), lambda i: (0, i, 0)),
            pl.BlockSpec((1, H), lambda i: (0, 0)),
            pl.BlockSpec((_BLK, 1), lambda i: (i, 0)),
            pl.BlockSpec((_BLK, 1), lambda i: (i, 0)),
            pl.BlockSpec((H, wout), lambda i: (0, 0)),
        ],
        out_specs=pl.BlockSpec((_BLK, wout), lambda i: (i, 0)),
        out_shape=jax.ShapeDtypeStruct((NPAD, wout), jnp.float32),
    )(p, b, deg_in, deg_out, w)


def _final_body(p_ref, b_ref, din_ref, o_ref):
    rs_in = lax.rsqrt(jnp.maximum(din_ref[...], 1.0))
    o_ref[...] = (p_ref[0] + p_ref[1]) * rs_in + b_ref[...]


def _final(p, b, deg_in):
    return pl.pallas_call(
        _final_body,
        grid=(_GRID,),
        in_specs=[
            pl.BlockSpec((NC, _BLK, CP), lambda i: (0, i, 0)),
            pl.BlockSpec((1, CP), lambda i: (0, 0)),
            pl.BlockSpec((_BLK, 1), lambda i: (i, 0)),
        ],
        out_specs=pl.BlockSpec((_BLK, CP), lambda i: (i, 0)),
        out_shape=jax.ShapeDtypeStruct((NPAD, CP), jnp.float32),
    )(p, b, deg_in)


def kernel(x, edge_index, W1, b1, W2, b2, W3, b3):
    xp = jnp.zeros((NPAD, D), jnp.float32).at[:N].set(x)
    ei4 = edge_index.reshape(2, NS, DCH // NS, KD)
    eidx = jnp.stack(
        [
            edge_index[0].reshape(NW, NCHK, KE),
            edge_index[1].reshape(NW, NCHK, KE),
        ],
        axis=2,
    ).reshape(NW * NCHK, 2, KE)
    W3p = jnp.pad(W3, ((0, 0), (0, CP - C)))
    b3p = jnp.pad(b3, (0, CP - C))

    deg = _degrees(ei4)
    deg_out = deg[0].reshape(NPAD, 1)
    deg_in = deg[1].reshape(NPAD, 1)

    h1 = _mm_scale(xp, W1, deg_out)
    p1 = _agg_h(h1, eidx)
    h2 = _combine_mm(p1, b1.reshape(1, H), deg_in, deg_out, W2)
    p2 = _agg_h(h2, eidx)
    h3 = _combine_mm(p2, b2.reshape(1, H), deg_in, deg_out, W3p)
    p3 = _agg_c(h3, eidx)
    out = _final(p3, b3p.reshape(1, CP), deg_in)
    return out[:N, :C]


# speedup vs baseline: 1.0224x; 1.0039x over previous
import functools

import jax
import jax.numpy as jnp
from jax import lax
from jax.experimental import pallas as pl
from jax.experimental.pallas import tpu as pltpu
from jax.experimental.pallas import tpu_sc as plsc

N = 10000
E = 320000
D = 128
H = 128
C = 40
CP = 128
NPAD = 10240
NC = 2
NS = 16
NW = NC * NS
LANES = 16

KD = 80
DCH = E // KD
KE = 125
ROWS_PER_TILE = NPAD // NS

_MESH = plsc.VectorSubcoreMesh(
    core_axis_name="c", subcore_axis_name="s", num_cores=NC, num_subcores=NS
)


def _zero_fill_2d(ref, rows, cols):
    zv = jnp.zeros((LANES,), jnp.float32)

    def body(i, _):
        r = i // (cols // LANES)
        col = (i % (cols // LANES)) * LANES
        ref[r, pl.ds(col, LANES)] = zv
        return 0

    lax.fori_loop(0, rows * (cols // LANES), body, 0)


@functools.partial(
    pl.kernel,
    out_type=jax.ShapeDtypeStruct((NC, 1, NPAD), jnp.float32),
    mesh=_MESH,
    scratch_types=[
        pltpu.VMEM_SHARED((NPAD,), jnp.float32),
        pltpu.VMEM((DCH // NS, KD), jnp.int32),
        pltpu.VMEM((KD,), jnp.float32),
        pltpu.VMEM((ROWS_PER_TILE,), jnp.float32),
        pltpu.SemaphoreType.DMA,
        pltpu.SemaphoreType.DMA,
    ],
)
def _degrees(ei_hbm, deg_hbm, acc, idx_v, ones_v, zb_v, sa, sb):
    c = lax.axis_index("c")
    s = lax.axis_index("s")

    def fill(i, _):
        ones_v[pl.ds(i * LANES, LANES)] = jnp.ones((LANES,), jnp.float32)
        return 0

    lax.fori_loop(0, KD // LANES, fill, 0)

    def fillz(i, _):
        zb_v[pl.ds(i * LANES, LANES)] = jnp.zeros((LANES,), jnp.float32)
        return 0

    lax.fori_loop(0, ROWS_PER_TILE // LANES, fillz, 0)

    pltpu.sync_copy(zb_v, acc.at[pl.ds(s * ROWS_PER_TILE, ROWS_PER_TILE)])
    plsc.subcore_barrier()

    nrows = DCH // NS
    pltpu.sync_copy(ei_hbm.at[c, s], idx_v)

    def sca_d(j, sem):
        return pltpu.make_async_copy(ones_v, acc.at[idx_v.at[j]], sem)

    sca_d(0, sa).start(add=True)

    def chunk(i2, _):
        j = 2 * i2 + 1
        sca_d(j, sb).start(add=True)
        sca_d(j - 1, sa).wait()
        sca_d(j + 1, sa).start(add=True)
        sca_d(j, sb).wait()
        return 0

    lax.fori_loop(0, (nrows - 2) // 2, chunk, 0)
    sca_d(nrows - 1, sb).start(add=True)
    sca_d(nrows - 2, sa).wait()
    sca_d(nrows - 1, sb).wait()
    plsc.subcore_barrier()

    pltpu.sync_copy(
        acc.at[pl.ds(s * ROWS_PER_TILE, ROWS_PER_TILE)],
        deg_hbm.at[c, 0, pl.ds(s * ROWS_PER_TILE, ROWS_PER_TILE)],
    )


NCHK = (E // NW) // KE
ZROWS = 40


def _make_agg(width):
    @functools.partial(
        pl.kernel,
        out_type=jax.ShapeDtypeStruct((NC, NPAD, width), jnp.float32),
        mesh=_MESH,
        scratch_types=[
            pltpu.VMEM_SHARED((NPAD, width), jnp.float32),
            pltpu.VMEM((KE, width), jnp.float32),
            pltpu.VMEM((KE, width), jnp.float32),
            pltpu.VMEM((2, KE), jnp.int32),
            pltpu.VMEM((2, KE), jnp.int32),
            pltpu.VMEM((2, KE), jnp.int32),
            pltpu.VMEM((2, KE), jnp.int32),
            pltpu.SemaphoreType.DMA,
            pltpu.SemaphoreType.DMA,
            pltpu.SemaphoreType.DMA,
            pltpu.SemaphoreType.DMA,
            pltpu.SemaphoreType.DMA,
        ],
    )
    def agg(h_hbm, eidx_hbm, out_hbm, acc, r0, r1, i0, i1, i2, i3,
            sg0, sg1, ss0, ss1, si):
        c = lax.axis_index("c")
        s = lax.axis_index("s")
        wid = c * NS + s
        rr = (r0, r1)
        ii = (i0, i1, i2, i3)
        sg = (sg0, sg1)
        ss = (ss0, ss1)

        def gat(iv, rv, sem):
            return pltpu.make_async_copy(h_hbm.at[iv.at[0]], rv, sem)

        def sca(rv, iv, sem):
            return pltpu.make_async_copy(rv, acc.at[iv.at[1]], sem)

        def ipre(j, iv):
            pltpu.make_async_copy(eidx_hbm.at[wid * NCHK + j], iv, si).start()

        def iwait(iv):
            pltpu.make_async_copy(eidx_hbm.at[0], iv, si).wait()

        _zero_fill_2d(r0, ZROWS, width)

        def zc(k, _):
            pltpu.sync_copy(
                r0.at[pl.ds(0, ZROWS), :],
                acc.at[pl.ds(s * ROWS_PER_TILE + k * ZROWS, ZROWS), :],
            )
            return 0

        lax.fori_loop(0, ROWS_PER_TILE // ZROWS, zc, 0)
        plsc.subcore_barrier()

        def emit(p, k, jn):
            iwait(ii[k])
            sca(rr[p], ii[k], ss[p]).wait()
            gat(ii[k], rr[p], sg[p]).start()
            gat(ii[k], rr[1 - p], sg[1 - p]).wait()
            sca(rr[1 - p], ii[(k - 1) % 4], ss[1 - p]).start(add=True)
            if jn is not None:
                ipre(jn, ii[(k + 1) % 4])

        pltpu.sync_copy(eidx_hbm.at[wid * NCHK], i0)
        gat(i0, r0, sg0).start()
        ipre(1, i1)
        iwait(i1)
        gat(i1, r1, sg1).start()
        gat(i1, r0, sg0).wait()
        sca(r0, i0, ss0).start(add=True)
        ipre(2, i2)

        def body(t, _):
            j = 4 * t + 2
            emit(0, 2, j + 1)
            emit(1, 3, j + 2)
            emit(0, 0, j + 3)
            emit(1, 1, j + 4)
            return 0

        lax.fori_loop(0, (NCHK - 4) // 4, body, 0)
        emit(0, 2, NCHK - 1)
        emit(1, 3, None)
        gat(i3, r1, sg1).wait()
        sca(r1, i3, ss1).start(add=True)
        sca(r0, i2, ss0).wait()
        sca(r1, i3, ss1).wait()
        plsc.subcore_barrier()

        pltpu.sync_copy(
            acc.at[pl.ds(s * ROWS_PER_TILE, ROWS_PER_TILE), :],
            out_hbm.at[c, pl.ds(s * ROWS_PER_TILE, ROWS_PER_TILE), :],
        )

    return agg


_agg_h = _make_agg(H)
_agg_c = _agg_h


_BLK = 1024
_GRID = NPAD // _BLK


def _mm_scale_body(x_ref, w_ref, dout_ref, o_ref):
    rs = lax.rsqrt(jnp.maximum(dout_ref[...], 1.0))
    o_ref[...] = jnp.dot(
        x_ref[...], w_ref[...], preferred_element_type=jnp.float32
    ) * rs


def _mm_scale(xp, w, deg_out):
    return pl.pallas_call(
        _mm_scale_body,
        grid=(_GRID,),
        in_specs=[
            pl.BlockSpec((_BLK, D), lambda i: (i, 0)),
            pl.BlockSpec((D, H), lambda i: (0, 0)),
            pl.BlockSpec((_BLK, 1), lambda i: (i, 0)),
        ],
        out_specs=pl.BlockSpec((_BLK, H), lambda i: (i, 0)),
        out_shape=jax.ShapeDtypeStruct((NPAD, H), jnp.float32),
    )(xp, w, deg_out)


def _combine_mm_body(p_ref, b_ref, din_ref, dout_ref, w_ref, o_ref):
    rs_in = lax.rsqrt(jnp.maximum(din_ref[...], 1.0))
    rs_out = lax.rsqrt(jnp.maximum(dout_ref[...], 1.0))
    h = (p_ref[0] + p_ref[1]) * rs_in + b_ref[...]
    h = jnp.maximum(h, 0.0)
    o_ref[...] = jnp.dot(
        h, w_ref[...], preferred_element_type=jnp.float32
    ) * rs_out


def _combine_mm(p, b, deg_in, deg_out, w):
    wout = w.shape[1]
    return pl.pallas_call(
        _combine_mm_body,
        grid=(_GRID,),
        in_specs=[
            pl.BlockSpec((NC, _BLK, H), lambda i: (0, i, 0)),
            pl.BlockSpec((1, H), lambda i: (0, 0)),
            pl.BlockSpec((_BLK, 1), lambda i: (i, 0)),
            pl.BlockSpec((_BLK, 1), lambda i: (i, 0)),
            pl.BlockSpec((H, wout), lambda i: (0, 0)),
        ],
        out_specs=pl.BlockSpec((_BLK, wout), lambda i: (i, 0)),
        out_shape=jax.ShapeDtypeStruct((NPAD, wout), jnp.float32),
    )(p, b, deg_in, deg_out, w)


def _final_body(p_ref, b_ref, din_ref, o_ref):
    rs_in = lax.rsqrt(jnp.maximum(din_ref[...], 1.0))
    o_ref[...] = (p_ref[0] + p_ref[1]) * rs_in + b_ref[...]


def _final(p, b, deg_in):
    return pl.pallas_call(
        _final_body,
        grid=(_GRID,),
        in_specs=[
            pl.BlockSpec((NC, _BLK, CP), lambda i: (0, i, 0)),
            pl.BlockSpec((1, CP), lambda i: (0, 0)),
            pl.BlockSpec((_BLK, 1), lambda i: (i, 0)),
        ],
        out_specs=pl.BlockSpec((_BLK, CP), lambda i: (i, 0)),
        out_shape=jax.ShapeDtypeStruct((NPAD, CP), jnp.float32),
    )(p, b, deg_in)


def kernel(x, edge_index, W1, b1, W2, b2, W3, b3):
    xp = jnp.zeros((NPAD, D), jnp.float32).at[:N].set(x)
    ei4 = edge_index.reshape(2, NS, DCH // NS, KD)
    eidx = jnp.stack(
        [
            edge_index[0].reshape(NW, NCHK, KE),
            edge_index[1].reshape(NW, NCHK, KE),
        ],
        axis=2,
    ).reshape(NW * NCHK, 2, KE)
    W3p = jnp.pad(W3, ((0, 0), (0, CP - C)))
    b3p = jnp.pad(b3, (0, CP - C))

    deg = _degrees(ei4)
    deg_out = deg[0].reshape(NPAD, 1)
    deg_in = deg[1].reshape(NPAD, 1)

    h1 = _mm_scale(xp, W1, deg_out)
    p1 = _agg_h(h1, eidx)
    h2 = _combine_mm(p1, b1.reshape(1, H), deg_in, deg_out, W2)
    p2 = _agg_h(h2, eidx)
    h3 = _combine_mm(p2, b2.reshape(1, H), deg_in, deg_out, W3p)
    p3 = _agg_c(h3, eidx)
    out = _final(p3, b3p.reshape(1, CP), deg_in)
    return out[:N, :C]
